# baseline (speedup 1.0000x reference)
import functools

import jax
import jax.numpy as jnp
from jax import lax
from jax.experimental import pallas as pl
from jax.experimental.pallas import tpu as pltpu
from jax.experimental.pallas import tpu_sc as plsc

_BATCH = 16384
_DIM = 64
_GRP = 16
_SLOTS = 32


def _two_tower_sc(u, p, user_emb, prod_emb):
    info = plsc.get_sparse_core_info()
    nw = info.num_cores * info.num_subcores
    b_per_w = _BATCH // nw
    n_grp = b_per_w // _GRP
    mesh = plsc.VectorSubcoreMesh(core_axis_name="c", subcore_axis_name="s")

    @functools.partial(
        pl.kernel,
        mesh=mesh,
        out_type=jax.ShapeDtypeStruct((_BATCH,), jnp.float32),
        compiler_params=pltpu.CompilerParams(use_tc_tiling_on_sc=True),
        scratch_types=[
            pltpu.VMEM((b_per_w + 16,), jnp.int32),
            pltpu.VMEM((b_per_w + 16,), jnp.int32),
            pltpu.VMEM((_SLOTS, 8, _DIM), jnp.float32),
            pltpu.VMEM((_SLOTS, 8, _DIM), jnp.float32),
            pltpu.VMEM((b_per_w,), jnp.float32),
            pltpu.SemaphoreType.DMA,
            pltpu.SemaphoreType.DMA,
        ],
    )
    def tile_task(u_hbm, p_hbm, ue_hbm, pe_hbm, out_hbm,
                  uidx_v, pidx_v, ublk_v, pblk_v, out_v, usem, psem):
        wid = lax.axis_index("s") * info.num_cores + lax.axis_index("c")
        base = wid * b_per_w

        pltpu.sync_copy(u_hbm.at[pl.ds(base, b_per_w)],
                        uidx_v.at[pl.ds(0, b_per_w)])
        pltpu.sync_copy(p_hbm.at[pl.ds(base, b_per_w)],
                        pidx_v.at[pl.ds(0, b_per_w)])

        iota16 = lax.iota(jnp.int32, 16)
        dnums = lax.GatherDimensionNumbers(
            offset_dims=(), collapsed_slice_dims=(0,), start_index_map=(0,))

        def lane_perm(x, idx):
            return lax.gather(
                x, idx[:, None], dimension_numbers=dnums, slice_sizes=(1,),
                mode=lax.GatherScatterMode.PROMISE_IN_BOUNDS)

        def fire(b):
            sl = b & (_SLOTS - 1)
            ru = uidx_v[pl.ds(b, 16)][0]
            rp = pidx_v[pl.ds(b, 16)][0]
            u0 = pl.multiple_of((ru >> 3) * 8, 8)
            p0 = pl.multiple_of((rp >> 3) * 8, 8)
            pltpu.async_copy(ue_hbm.at[pl.ds(u0, 8)], ublk_v.at[sl], usem)
            pltpu.async_copy(pe_hbm.at[pl.ds(p0, 8)], pblk_v.at[sl], psem)

        def drain():
            pltpu.make_async_copy(
                ue_hbm.at[pl.ds(0, 8)], ublk_v.at[0], usem).wait()
            pltpu.make_async_copy(
                pe_hbm.at[pl.ds(0, 8)], pblk_v.at[0], psem).wait()

        def consume_group(t):
            acc = jnp.zeros((16,), jnp.float32)
            for l in range(16):
                b = t * _GRP + l
                sl = b & (_SLOTS - 1)
                ru = uidx_v[pl.ds(b, 16)][0]
                rp = pidx_v[pl.ds(b, 16)][0]
                rmu = ru & 7
                rmp = rp & 7
                prod = jnp.zeros((16,), jnp.float32)
                for k in range(_DIM // 16):
                    prod = prod + (ublk_v[sl, rmu, pl.ds(k * 16, 16)]
                                   * pblk_v[sl, rmp, pl.ds(k * 16, 16)])
                for sh in (8, 4, 2, 1):
                    prod = prod + lane_perm(prod, iota16 ^ sh)
                acc = jnp.where(iota16 == l, prod, acc)
            out_v[pl.ds(t * _GRP, 16)] = 1.0 / (1.0 + jnp.exp(-acc))

        def fire_group(t):
            for l in range(16):
                fire(t * _GRP + l)

        def drain_group():
            for _ in range(16):
                drain()

        fire_group(0)
        drain_group()

        def steady(t, carry):
            fire_group(t + 1)
            consume_group(t)
            drain_group()
            return carry

        lax.fori_loop(0, n_grp - 1, steady, 0)
        consume_group(n_grp - 1)

        pltpu.sync_copy(out_v, out_hbm.at[pl.ds(base, b_per_w)])

    return tile_task(u, p, user_emb, prod_emb)


def kernel(u, p, user_emb, prod_emb):
    return _two_tower_sc(u, p, user_emb, prod_emb)
